# baseline (speedup 1.0000x reference)
import functools

import jax
import jax.numpy as jnp
from jax import lax
from jax.experimental import pallas as pl
from jax.experimental.pallas import tpu as pltpu
from jax.experimental.pallas import tpu_sc as plsc

_NC = 2
_NS = 16
_NW = _NC * _NS
_LANES = 16


def _sc_gather(user_emb4, item_emb4, user_id, item_id):
    B = user_id.shape[0]
    W = user_emb4.shape[1]
    b_per_w = B // _NW
    mesh = plsc.VectorSubcoreMesh(core_axis_name="c", subcore_axis_name="s")

    @functools.partial(
        pl.kernel,
        mesh=mesh,
        out_type=(
            jax.ShapeDtypeStruct((B, W), jnp.float32),
            jax.ShapeDtypeStruct((B, W), jnp.float32),
        ),
        scratch_types=[
            pltpu.VMEM((b_per_w,), jnp.int32),
            pltpu.VMEM((b_per_w,), jnp.int32),
            pltpu.VMEM((b_per_w // 2, W), jnp.float32),
            pltpu.VMEM((b_per_w // 2, W), jnp.float32),
            pltpu.SemaphoreType.DMA,
            pltpu.SemaphoreType.DMA,
        ],
    )
    def gather_kernel(uemb_hbm, iemb_hbm, uid_hbm, iid_hbm, u_out, v_out,
                      uidx_v, iidx_v, urows_v, irows_v, sem_u, sem_i):
        wid = lax.axis_index("s") * _NC + lax.axis_index("c")
        base = wid * b_per_w
        half = b_per_w // 2
        pltpu.sync_copy(uid_hbm.at[pl.ds(base, b_per_w)], uidx_v)
        pltpu.sync_copy(iid_hbm.at[pl.ds(base, b_per_w)], iidx_v)

        @pl.loop(0, b_per_w, step=_LANES)
        def _(i):
            slc = pl.ds(i, _LANES)
            uidx_v[slc] = lax.shift_right_logical(uidx_v[slc], 2)
            iidx_v[slc] = lax.shift_right_logical(iidx_v[slc], 2)

        for c in range(2):
            cu = pltpu.async_copy(uemb_hbm.at[uidx_v.at[pl.ds(c * half, half)]],
                                  urows_v, sem_u)
            ci = pltpu.async_copy(iemb_hbm.at[iidx_v.at[pl.ds(c * half, half)]],
                                  irows_v, sem_i)
            cu.wait()
            ci.wait()
            pltpu.sync_copy(urows_v, u_out.at[pl.ds(base + c * half, half)])
            pltpu.sync_copy(irows_v, v_out.at[pl.ds(base + c * half, half)])

    return gather_kernel(user_emb4, item_emb4, user_id, item_id)


def _tc_head(u4, v4, uid, iid, W1u, W1v, b1, W2, b2, W3, bias0, D):
    B = u4.shape[0]
    W = u4.shape[1]
    blk = 4096
    dn = (((1,), (1,)), ((), ()))

    def body(u4_ref, v4_ref, uid_ref, iid_ref, w1u_ref, w1v_ref, b1_ref,
             w2_ref, b2_ref, w3_ref, bias_ref, o_ref):
        usub = jnp.bitwise_and(uid_ref[...], 3)
        isub = jnp.bitwise_and(iid_ref[...], 3)
        uu = jnp.zeros((blk, D), jnp.float32)
        vv = jnp.zeros((blk, D), jnp.float32)
        for k in range(W // D):
            uu = uu + jnp.where(usub == k,
                                u4_ref[:, k * D:(k + 1) * D], 0.0)
            vv = vv + jnp.where(isub == k,
                                v4_ref[:, k * D:(k + 1) * D], 0.0)
        fm = jnp.sum(uu + vv + uu * vv, axis=1)
        h1 = jnp.maximum(
            lax.dot_general(uu, w1u_ref[...], dn,
                            preferred_element_type=jnp.float32)
            + lax.dot_general(vv, w1v_ref[...], dn,
                              preferred_element_type=jnp.float32)
            + b1_ref[...], 0.0)
        h2 = jnp.maximum(
            lax.dot_general(h1, w2_ref[...], dn,
                            preferred_element_type=jnp.float32)
            + b2_ref[...], 0.0)
        deep = lax.dot_general(h2, w3_ref[...], dn,
                               preferred_element_type=jnp.float32)[:, 0]
        o_ref[...] = fm + deep + bias_ref[0, 0]

    H1 = W1u.shape[0]
    H2 = W2.shape[0]
    return pl.pallas_call(
        body,
        grid=(B // blk,),
        in_specs=[
            pl.BlockSpec((blk, W), lambda i: (i, 0)),
            pl.BlockSpec((blk, W), lambda i: (i, 0)),
            pl.BlockSpec((blk, 1), lambda i: (i, 0)),
            pl.BlockSpec((blk, 1), lambda i: (i, 0)),
            pl.BlockSpec((H1, D), lambda i: (0, 0)),
            pl.BlockSpec((H1, D), lambda i: (0, 0)),
            pl.BlockSpec((1, H1), lambda i: (0, 0)),
            pl.BlockSpec((H2, H1), lambda i: (0, 0)),
            pl.BlockSpec((1, H2), lambda i: (0, 0)),
            pl.BlockSpec((1, H2), lambda i: (0, 0)),
            pl.BlockSpec((1, 1), lambda i: (0, 0)),
        ],
        out_specs=pl.BlockSpec((blk,), lambda i: (i,)),
        out_shape=jax.ShapeDtypeStruct((B,), jnp.float32),
    )(u4, v4, uid.reshape(-1, 1), iid.reshape(-1, 1), W1u, W1v, b1, W2,
      b2, W3, bias0)


def kernel(user_id, item_id, user_emb, item_emb, fm_bias, W1, b1, W2, b2,
           W3, b3):
    D = user_emb.shape[1]
    uid = user_id.astype(jnp.int32)
    iid = item_id.astype(jnp.int32)
    u4, v4 = _sc_gather(user_emb.reshape(-1, 4 * D),
                        item_emb.reshape(-1, 4 * D), uid, iid)
    W1u = W1[:, :D]
    W1v = W1[:, D:]
    bias0 = (fm_bias + b3).reshape(1, 1)
    return _tc_head(u4, v4, uid, iid, W1u, W1v, b1.reshape(1, -1), W2,
                    b2.reshape(1, -1), W3, bias0, D)
